# baseline (speedup 1.0000x reference)
import jax
import jax.numpy as jnp
from jax import lax
from jax.experimental import pallas as pl
from jax.experimental.pallas import tpu as pltpu
from jax.experimental.pallas import tpu_sc as plsc

N_ROWS = 128
N_COLS = 32768
L = 16
NC = 2
NS = 16
NW = NC * NS

TC_ROWS = 64
SC_ROWS = N_ROWS - TC_ROWS
ROWS_PER_W = SC_ROWS // NW
TC_BLK = 8

CHUNKS = N_COLS // L
BLK_CHUNKS = 64
NBLK = CHUNKS // BLK_CHUNKS
ACC = 8
NBUF = 2
IMAX = jnp.iinfo(jnp.int32).max

_mesh = plsc.VectorSubcoreMesh(core_axis_name="c", subcore_axis_name="s",
                               num_cores=NC, num_subcores=NS)

_SCRATCH = [
    pltpu.VMEM((N_COLS,), jnp.float32),
    pltpu.VMEM((N_COLS,), jnp.float32),
    pltpu.VMEM((NBLK * L,), jnp.float32),
    pltpu.VMEM((L,), jnp.int32),
    pltpu.VMEM((L,), jnp.float32),
    pltpu.VMEM((L,), jnp.int32),
    pltpu.SemaphoreType.DMA,
    pltpu.SemaphoreType.DMA,
]


def _argmax_body(x_hbm, out_hbm, buf0, buf1, blkmax, resv, tmpv, tmpi,
                 sem0, sem1):
    c = lax.axis_index("c")
    s = lax.axis_index("s")
    w = c * NS + s
    row0 = TC_ROWS + w * ROWS_PER_W
    iota = lax.iota(jnp.int32, L)
    neg_inf = jnp.full((L,), -jnp.inf, jnp.float32)
    imax_v = jnp.full((L,), IMAX, jnp.int32)

    bufs = (buf0, buf1)
    sems = (sem0, sem1)
    descs = [None] * NBUF

    def start_row(r):
        descs[r % NBUF] = pltpu.async_copy(
            x_hbm.at[row0 + r], bufs[r % NBUF], sems[r % NBUF])

    for r in range(min(NBUF, ROWS_PER_W)):
        start_row(r)

    results = jnp.zeros((L,), jnp.int32)
    for r in range(ROWS_PER_W):
        descs[r % NBUF].wait()
        cur = bufs[r % NBUF]

        @plsc.parallel_loop(0, NBLK, carry=neg_inf)
        def gmax(b, gacc, cur=cur):
            base = b * (BLK_CHUNKS * L)

            @plsc.parallel_loop(0, BLK_CHUNKS, step=ACC, unroll=4,
                                carry=(neg_inf,) * ACC)
            def accs(i, ms):
                return tuple(
                    jnp.maximum(m, cur[pl.ds(base + (i + a) * L, L)])
                    for a, m in enumerate(ms))

            t0 = jnp.maximum(jnp.maximum(accs[0], accs[1]),
                             jnp.maximum(accs[2], accs[3]))
            t1 = jnp.maximum(jnp.maximum(accs[4], accs[5]),
                             jnp.maximum(accs[6], accs[7]))
            bm = jnp.maximum(t0, t1)
            blkmax[pl.ds(b * L, L)] = bm
            return jnp.maximum(gacc, bm)

        if r + NBUF < ROWS_PER_W:
            start_row(r + NBUF)

        mx = gmax
        for k in (8, 4, 2, 1):
            tmpv[...] = mx
            mx = jnp.maximum(mx, plsc.load_gather(tmpv, [iota ^ k]))

        @plsc.parallel_loop(0, NBLK, unroll=4, carry=imax_v)
        def firstb(i, fb):
            v = blkmax[pl.ds(i * L, L)]
            return jnp.minimum(fb, jnp.where(v == mx,
                                             jnp.full((L,), i, jnp.int32),
                                             imax_v))

        fb = firstb
        for k in (8, 4, 2, 1):
            tmpi[...] = fb
            fb = jnp.minimum(fb, plsc.load_gather(tmpi, [iota ^ k]))
        bstar = fb[0]

        base = bstar * (BLK_CHUNKS * L)

        @plsc.parallel_loop(0, BLK_CHUNKS, unroll=2,
                            carry=(neg_inf, jnp.zeros((L,), jnp.int32)))
        def scan(i, cr, cur=cur):
            best, bidx = cr
            v = cur[pl.ds(base + i * L, L)]
            m = v > best
            return (jnp.where(m, v, best),
                    jnp.where(m, jnp.full((L,), i, jnp.int32), bidx))

        best, ix = scan[0], (bstar * BLK_CHUNKS + scan[1]) * L + iota
        for k in (8, 4, 2, 1):
            tmpv[...] = best
            tmpi[...] = ix
            v2 = plsc.load_gather(tmpv, [iota ^ k])
            i2 = plsc.load_gather(tmpi, [iota ^ k])
            m = (v2 > best) | ((v2 == best) & (i2 < ix))
            best = jnp.where(m, v2, best)
            ix = jnp.where(m, i2, ix)
        results = jnp.where(iota == r, ix, results)

    resv[...] = results
    pltpu.sync_copy(resv, out_hbm.at[w])


_argmax_sc = pl.kernel(
    _argmax_body,
    out_type=jax.ShapeDtypeStruct((NW, L), jnp.int32),
    mesh=_mesh,
    compiler_params=pltpu.CompilerParams(needs_layout_passes=False),
    scratch_types=_SCRATCH,
)


def _tc_body(x_ref, o_ref):
    v = x_ref[...]
    m = jnp.max(v, axis=1, keepdims=True)
    idx = lax.broadcasted_iota(jnp.int32, v.shape, 1)
    cand = jnp.where(v == m, idx, IMAX)
    o_ref[0, 0, :] = jnp.min(cand, axis=1)


_tc_argmax = pl.pallas_call(
    _tc_body,
    grid=(TC_ROWS // TC_BLK,),
    in_specs=[pl.BlockSpec((TC_BLK, N_COLS), lambda i: (i, 0))],
    out_specs=pl.BlockSpec((1, 1, TC_BLK), lambda i: (i, 0, 0)),
    out_shape=jax.ShapeDtypeStruct((TC_ROWS // TC_BLK, 1, TC_BLK), jnp.int32),
)


def kernel(x):
    board = _argmax_sc(x)
    tc = _tc_argmax(x[:TC_ROWS])
    return jnp.concatenate(
        [tc.reshape(TC_ROWS), board[:, :ROWS_PER_W].reshape(SC_ROWS)])


# speedup vs baseline: 1.0540x; 1.0540x over previous
import jax
import jax.numpy as jnp
from jax import lax
from jax.experimental import pallas as pl
from jax.experimental.pallas import tpu as pltpu
from jax.experimental.pallas import tpu_sc as plsc

N_ROWS = 128
N_COLS = 32768
L = 16
NC = 2
NS = 16
NW = NC * NS

TC_ROWS = 64
SC_ROWS = N_ROWS - TC_ROWS
ROWS_PER_W = SC_ROWS // NW
TC_BLK = 8

CHUNKS = N_COLS // L
BLK_CHUNKS = 64
NBLK = CHUNKS // BLK_CHUNKS
ACC = 8
NBUF = 2
IMAX = jnp.iinfo(jnp.int32).max

_mesh = plsc.VectorSubcoreMesh(core_axis_name="c", subcore_axis_name="s",
                               num_cores=NC, num_subcores=NS)

_SCRATCH = [
    pltpu.VMEM((N_COLS,), jnp.float32),
    pltpu.VMEM((N_COLS,), jnp.float32),
    pltpu.VMEM((NBLK * L,), jnp.float32),
    pltpu.VMEM((L,), jnp.int32),
    pltpu.VMEM((L,), jnp.float32),
    pltpu.VMEM((L,), jnp.int32),
    pltpu.SemaphoreType.DMA,
    pltpu.SemaphoreType.DMA,
]


def _argmax_body(x_hbm, tca_hbm, out_hbm, pass_hbm, buf0, buf1, blkmax, resv,
                 tmpv, tmpi, sem0, sem1):
    c = lax.axis_index("c")
    s = lax.axis_index("s")
    w = c * NS + s
    row0 = TC_ROWS + w * ROWS_PER_W
    iota = lax.iota(jnp.int32, L)
    neg_inf = jnp.full((L,), -jnp.inf, jnp.float32)
    imax_v = jnp.full((L,), IMAX, jnp.int32)

    bufs = (buf0, buf1)
    sems = (sem0, sem1)
    descs = [None] * NBUF

    def start_row(r):
        descs[r % NBUF] = pltpu.async_copy(
            x_hbm.at[row0 + r], bufs[r % NBUF], sems[r % NBUF])

    for r in range(min(NBUF, ROWS_PER_W)):
        start_row(r)

    results = jnp.zeros((L,), jnp.int32)
    for r in range(ROWS_PER_W):
        descs[r % NBUF].wait()
        cur = bufs[r % NBUF]

        @plsc.parallel_loop(0, NBLK, carry=neg_inf)
        def gmax(b, gacc, cur=cur):
            base = b * (BLK_CHUNKS * L)

            @plsc.parallel_loop(0, BLK_CHUNKS, step=ACC, unroll=4,
                                carry=(neg_inf,) * ACC)
            def accs(i, ms):
                return tuple(
                    jnp.maximum(m, cur[pl.ds(base + (i + a) * L, L)])
                    for a, m in enumerate(ms))

            t0 = jnp.maximum(jnp.maximum(accs[0], accs[1]),
                             jnp.maximum(accs[2], accs[3]))
            t1 = jnp.maximum(jnp.maximum(accs[4], accs[5]),
                             jnp.maximum(accs[6], accs[7]))
            bm = jnp.maximum(t0, t1)
            blkmax[pl.ds(b * L, L)] = bm
            return jnp.maximum(gacc, bm)

        if r + NBUF < ROWS_PER_W:
            start_row(r + NBUF)

        mx = gmax
        for k in (8, 4, 2, 1):
            tmpv[...] = mx
            mx = jnp.maximum(mx, plsc.load_gather(tmpv, [iota ^ k]))

        @plsc.parallel_loop(0, NBLK, unroll=4, carry=imax_v)
        def firstb(i, fb):
            v = blkmax[pl.ds(i * L, L)]
            return jnp.minimum(fb, jnp.where(v == mx,
                                             jnp.full((L,), i, jnp.int32),
                                             imax_v))

        fb = firstb
        for k in (8, 4, 2, 1):
            tmpi[...] = fb
            fb = jnp.minimum(fb, plsc.load_gather(tmpi, [iota ^ k]))
        bstar = fb[0]

        base = bstar * (BLK_CHUNKS * L)

        @plsc.parallel_loop(0, BLK_CHUNKS, unroll=2,
                            carry=(neg_inf, jnp.zeros((L,), jnp.int32)))
        def scan(i, cr, cur=cur):
            best, bidx = cr
            v = cur[pl.ds(base + i * L, L)]
            m = v > best
            return (jnp.where(m, v, best),
                    jnp.where(m, jnp.full((L,), i, jnp.int32), bidx))

        best, ix = scan[0], (bstar * BLK_CHUNKS + scan[1]) * L + iota
        for k in (8, 4, 2, 1):
            tmpv[...] = best
            tmpi[...] = ix
            v2 = plsc.load_gather(tmpv, [iota ^ k])
            i2 = plsc.load_gather(tmpi, [iota ^ k])
            m = (v2 > best) | ((v2 == best) & (i2 < ix))
            best = jnp.where(m, v2, best)
            ix = jnp.where(m, i2, ix)
        results = jnp.where(iota == r, ix, results)

    resv[...] = results
    pltpu.sync_copy(resv, out_hbm.at[w])

    @pl.when(w == 0)
    def _passthrough():
        pltpu.sync_copy(tca_hbm.at[pl.ds(0, L)], tmpi)
        pltpu.sync_copy(tmpi, pass_hbm.at[0])
        pltpu.sync_copy(tca_hbm.at[pl.ds(L, L)], tmpi)
        pltpu.sync_copy(tmpi, pass_hbm.at[1])


_argmax_sc = pl.kernel(
    _argmax_body,
    out_type=[jax.ShapeDtypeStruct((NW, L), jnp.int32),
              jax.ShapeDtypeStruct((2, L), jnp.int32)],
    mesh=_mesh,
    compiler_params=pltpu.CompilerParams(needs_layout_passes=False),
    scratch_types=_SCRATCH,
)


def _tc_body(x_ref, o_ref):
    v = x_ref[...]
    m = jnp.max(v, axis=1, keepdims=True)
    idx = lax.broadcasted_iota(jnp.int32, v.shape, 1)
    cand = jnp.where(v == m, idx, IMAX)
    o_ref[0, 0, :] = jnp.min(cand, axis=1)


TCA_ROWS = TC_ROWS // 2
TCB_ROWS = TC_ROWS - TCA_ROWS


def _make_tc(row_lo, rows):
    return pl.pallas_call(
        _tc_body,
        grid=(rows // TC_BLK,),
        in_specs=[pl.BlockSpec((TC_BLK, N_COLS),
                               lambda i: (i + row_lo // TC_BLK, 0))],
        out_specs=pl.BlockSpec((1, 1, TC_BLK), lambda i: (i, 0, 0)),
        out_shape=jax.ShapeDtypeStruct((rows // TC_BLK, 1, TC_BLK),
                                       jnp.int32),
    )


_tc_argmax_a = _make_tc(0, TCA_ROWS)
_tc_argmax_b = _make_tc(TCA_ROWS, TCB_ROWS)


def kernel(x):
    tca = _tc_argmax_a(x).reshape(TCA_ROWS)
    board, tca_pass = _argmax_sc(x, tca)
    tcb = _tc_argmax_b(x)
    return jnp.concatenate(
        [tca_pass.reshape(TCA_ROWS), tcb.reshape(TCB_ROWS),
         board[:, :ROWS_PER_W].reshape(SC_ROWS)])


# speedup vs baseline: 1.2917x; 1.2256x over previous
import jax
import jax.numpy as jnp
from jax import lax
from jax.experimental import pallas as pl
from jax.experimental.pallas import tpu as pltpu
from jax.experimental.pallas import tpu_sc as plsc

N_ROWS = 128
N_COLS = 32768
L = 16
NC = 2
NS = 16
NW = NC * NS

TC_ROWS = 64
SC_ROWS = N_ROWS - TC_ROWS
ROWS_PER_W = SC_ROWS // NW
TC_BLK = 16

CHUNKS = N_COLS // L
BLK_CHUNKS = 64
NBLK = CHUNKS // BLK_CHUNKS
ACC = 8
NBUF = 2
IMAX = jnp.iinfo(jnp.int32).max

_mesh = plsc.VectorSubcoreMesh(core_axis_name="c", subcore_axis_name="s",
                               num_cores=NC, num_subcores=NS)

_SCRATCH = [
    pltpu.VMEM((N_COLS,), jnp.float32),
    pltpu.VMEM((N_COLS,), jnp.float32),
    pltpu.VMEM((NBLK * L,), jnp.float32),
    pltpu.VMEM((L,), jnp.int32),
    pltpu.VMEM((L,), jnp.float32),
    pltpu.VMEM((L,), jnp.int32),
    pltpu.SemaphoreType.DMA,
    pltpu.SemaphoreType.DMA,
]


def _argmax_body(x_hbm, out_hbm, buf0, buf1, blkmax, resv, tmpv, tmpi,
                 sem0, sem1):
    c = lax.axis_index("c")
    s = lax.axis_index("s")
    w = c * NS + s
    row0 = TC_ROWS + w * ROWS_PER_W
    iota = lax.iota(jnp.int32, L)
    neg_inf = jnp.full((L,), -jnp.inf, jnp.float32)
    imax_v = jnp.full((L,), IMAX, jnp.int32)

    bufs = (buf0, buf1)
    sems = (sem0, sem1)
    descs = [None] * NBUF

    def start_row(r):
        descs[r % NBUF] = pltpu.async_copy(
            x_hbm.at[row0 + r], bufs[r % NBUF], sems[r % NBUF])

    for r in range(min(NBUF, ROWS_PER_W)):
        start_row(r)

    results = jnp.zeros((L,), jnp.int32)
    for r in range(ROWS_PER_W):
        descs[r % NBUF].wait()
        cur = bufs[r % NBUF]

        @plsc.parallel_loop(0, NBLK, carry=neg_inf)
        def gmax(b, gacc, cur=cur):
            base = b * (BLK_CHUNKS * L)

            @plsc.parallel_loop(0, BLK_CHUNKS, step=ACC, unroll=4,
                                carry=(neg_inf,) * ACC)
            def accs(i, ms):
                return tuple(
                    jnp.maximum(m, cur[pl.ds(base + (i + a) * L, L)])
                    for a, m in enumerate(ms))

            t0 = jnp.maximum(jnp.maximum(accs[0], accs[1]),
                             jnp.maximum(accs[2], accs[3]))
            t1 = jnp.maximum(jnp.maximum(accs[4], accs[5]),
                             jnp.maximum(accs[6], accs[7]))
            bm = jnp.maximum(t0, t1)
            blkmax[pl.ds(b * L, L)] = bm
            return jnp.maximum(gacc, bm)

        if r + NBUF < ROWS_PER_W:
            start_row(r + NBUF)

        mx = gmax
        for k in (8, 4, 2, 1):
            tmpv[...] = mx
            mx = jnp.maximum(mx, plsc.load_gather(tmpv, [iota ^ k]))

        @plsc.parallel_loop(0, NBLK, unroll=4, carry=imax_v)
        def firstb(i, fb):
            v = blkmax[pl.ds(i * L, L)]
            return jnp.minimum(fb, jnp.where(v == mx,
                                             jnp.full((L,), i, jnp.int32),
                                             imax_v))

        fb = firstb
        for k in (8, 4, 2, 1):
            tmpi[...] = fb
            fb = jnp.minimum(fb, plsc.load_gather(tmpi, [iota ^ k]))
        bstar = fb[0]

        base = bstar * (BLK_CHUNKS * L)

        @plsc.parallel_loop(0, BLK_CHUNKS, unroll=2,
                            carry=(neg_inf, jnp.zeros((L,), jnp.int32)))
        def scan(i, cr, cur=cur):
            best, bidx = cr
            v = cur[pl.ds(base + i * L, L)]
            m = v > best
            return (jnp.where(m, v, best),
                    jnp.where(m, jnp.full((L,), i, jnp.int32), bidx))

        best, ix = scan[0], (bstar * BLK_CHUNKS + scan[1]) * L + iota
        for k in (8, 4, 2, 1):
            tmpv[...] = best
            tmpi[...] = ix
            v2 = plsc.load_gather(tmpv, [iota ^ k])
            i2 = plsc.load_gather(tmpi, [iota ^ k])
            m = (v2 > best) | ((v2 == best) & (i2 < ix))
            best = jnp.where(m, v2, best)
            ix = jnp.where(m, i2, ix)
        results = jnp.where(iota == r, ix, results)

    resv[...] = results
    pltpu.sync_copy(resv, out_hbm.at[w])


_argmax_sc = pl.kernel(
    _argmax_body,
    out_type=jax.ShapeDtypeStruct((NW, L), jnp.int32),
    mesh=_mesh,
    compiler_params=pltpu.CompilerParams(needs_layout_passes=False),
    scratch_types=_SCRATCH,
)


def _tc_body(x_ref, o_ref):
    v = x_ref[...]
    m = jnp.max(v, axis=1, keepdims=True)
    idx = lax.broadcasted_iota(jnp.int32, v.shape, 1)
    cand = jnp.where(v == m, idx, IMAX)
    o_ref[0, 0, :] = jnp.min(cand, axis=1)


_tc_argmax = pl.pallas_call(
    _tc_body,
    grid=(TC_ROWS // TC_BLK,),
    in_specs=[pl.BlockSpec((TC_BLK, N_COLS), lambda i: (i, 0))],
    out_specs=pl.BlockSpec((1, 1, TC_BLK), lambda i: (i, 0, 0)),
    out_shape=jax.ShapeDtypeStruct((TC_ROWS // TC_BLK, 1, TC_BLK), jnp.int32),
)


def kernel(x):
    board = _argmax_sc(x)
    tc = _tc_argmax(x)
    return jnp.concatenate(
        [tc.reshape(TC_ROWS), board[:, :ROWS_PER_W].reshape(SC_ROWS)])


# speedup vs baseline: 1.3022x; 1.0081x over previous
import jax
import jax.numpy as jnp
from jax import lax
from jax.experimental import pallas as pl
from jax.experimental.pallas import tpu as pltpu
from jax.experimental.pallas import tpu_sc as plsc

N_ROWS = 128
N_COLS = 32768
L = 16
NC = 2
NS = 16
NW = NC * NS

TC_ROWS = 64
SC_ROWS = N_ROWS - TC_ROWS
ROWS_PER_W = SC_ROWS // NW
TC_BLK = 16

CHUNKS = N_COLS // L
BLK_CHUNKS = 64
NBLK = CHUNKS // BLK_CHUNKS
ACC = 8
NBUF = 2
IMAX = jnp.iinfo(jnp.int32).max

_mesh = plsc.VectorSubcoreMesh(core_axis_name="c", subcore_axis_name="s",
                               num_cores=NC, num_subcores=NS)

_SCRATCH = [
    pltpu.VMEM((N_COLS,), jnp.float32),
    pltpu.VMEM((N_COLS,), jnp.float32),
    pltpu.VMEM((NBLK * L,), jnp.float32),
    pltpu.VMEM((L,), jnp.int32),
    pltpu.VMEM((L,), jnp.float32),
    pltpu.VMEM((L,), jnp.int32),
    pltpu.SemaphoreType.DMA,
    pltpu.SemaphoreType.DMA,
]


def _argmax_body(x_hbm, out_hbm, buf0, buf1, blkmax, resv, tmpv, tmpi,
                 sem0, sem1):
    c = lax.axis_index("c")
    s = lax.axis_index("s")
    w = c * NS + s
    row0 = TC_ROWS + w * ROWS_PER_W
    iota = lax.iota(jnp.int32, L)
    neg_inf = jnp.full((L,), -jnp.inf, jnp.float32)
    imax_v = jnp.full((L,), IMAX, jnp.int32)

    bufs = (buf0, buf1)
    sems = (sem0, sem1)
    descs = [None] * NBUF

    def start_row(r):
        descs[r % NBUF] = pltpu.async_copy(
            x_hbm.at[row0 + r], bufs[r % NBUF], sems[r % NBUF])

    for r in range(min(NBUF, ROWS_PER_W)):
        start_row(r)

    results = jnp.zeros((L,), jnp.int32)
    for r in range(ROWS_PER_W):
        descs[r % NBUF].wait()
        cur = bufs[r % NBUF]

        @plsc.parallel_loop(0, NBLK, carry=neg_inf)
        def gmax(b, gacc, cur=cur):
            base = b * (BLK_CHUNKS * L)

            @plsc.parallel_loop(0, BLK_CHUNKS, step=ACC, unroll=4,
                                carry=(neg_inf,) * ACC)
            def accs(i, ms):
                return tuple(
                    jnp.maximum(m, cur[pl.ds(base + (i + a) * L, L)])
                    for a, m in enumerate(ms))

            t0 = jnp.maximum(jnp.maximum(accs[0], accs[1]),
                             jnp.maximum(accs[2], accs[3]))
            t1 = jnp.maximum(jnp.maximum(accs[4], accs[5]),
                             jnp.maximum(accs[6], accs[7]))
            bm = jnp.maximum(t0, t1)
            blkmax[pl.ds(b * L, L)] = bm
            return jnp.maximum(gacc, bm)

        if r + NBUF < ROWS_PER_W:
            start_row(r + NBUF)

        mx = gmax
        for k in (8, 4, 2, 1):
            tmpv[...] = mx
            mx = jnp.maximum(mx, plsc.load_gather(tmpv, [iota ^ k]))

        @plsc.parallel_loop(0, NBLK, unroll=4, carry=imax_v)
        def firstb(i, fb):
            v = blkmax[pl.ds(i * L, L)]
            return jnp.minimum(fb, jnp.where(v == mx,
                                             jnp.full((L,), i, jnp.int32),
                                             imax_v))

        fb = firstb
        for k in (8, 4, 2, 1):
            tmpi[...] = fb
            fb = jnp.minimum(fb, plsc.load_gather(tmpi, [iota ^ k]))
        bstar = fb[0]

        base = bstar * (BLK_CHUNKS * L)

        @plsc.parallel_loop(0, BLK_CHUNKS, unroll=2,
                            carry=(neg_inf, jnp.zeros((L,), jnp.int32)))
        def scan(i, cr, cur=cur):
            best, bidx = cr
            v = cur[pl.ds(base + i * L, L)]
            m = v > best
            return (jnp.where(m, v, best),
                    jnp.where(m, jnp.full((L,), i, jnp.int32), bidx))

        best, ix = scan[0], (bstar * BLK_CHUNKS + scan[1]) * L + iota
        for k in (8, 4, 2, 1):
            tmpv[...] = best
            tmpi[...] = ix
            v2 = plsc.load_gather(tmpv, [iota ^ k])
            i2 = plsc.load_gather(tmpi, [iota ^ k])
            m = (v2 > best) | ((v2 == best) & (i2 < ix))
            best = jnp.where(m, v2, best)
            ix = jnp.where(m, i2, ix)
        results = jnp.where(iota == r, ix, results)

    resv[...] = results
    pltpu.sync_copy(resv, out_hbm.at[w])


_argmax_sc = pl.kernel(
    _argmax_body,
    out_type=jax.ShapeDtypeStruct((NW, L), jnp.int32),
    mesh=_mesh,
    compiler_params=pltpu.CompilerParams(needs_layout_passes=False),
    scratch_types=_SCRATCH,
)


def _tc_body(x_ref, o_ref):
    v = x_ref[...]
    m = jnp.max(v, axis=1, keepdims=True)
    idx = lax.broadcasted_iota(jnp.int32, v.shape, 1)
    cand = jnp.where(v == m, idx, IMAX)
    res = jnp.min(cand, axis=1, keepdims=True).astype(jnp.float32)
    o_ref[...] = jnp.broadcast_to(res, (TC_BLK, 128))


_tc_argmax = pl.pallas_call(
    _tc_body,
    grid=(TC_ROWS // TC_BLK,),
    in_specs=[pl.BlockSpec((TC_BLK, N_COLS), lambda i: (i, 0))],
    out_specs=pl.BlockSpec((TC_BLK, 128), lambda i: (i, 0)),
    out_shape=jax.ShapeDtypeStruct((TC_ROWS, 128), jnp.float32),
)


def kernel(x):
    board = _argmax_sc(x)
    tc = _tc_argmax(x)
    return jnp.concatenate(
        [tc[:, 0].astype(jnp.int32), board[:, :ROWS_PER_W].reshape(SC_ROWS)])
